# Initial kernel scaffold; baseline (speedup 1.0000x reference)
import jax
import jax.numpy as jnp
from jax.experimental import pallas as pl


def kernel(xyz, xyz_normalized, params):
    raise NotImplementedError("write your pallas kernel here")


# speedup vs baseline: 11.5832x; 11.5832x over previous
import functools

import jax
import jax.numpy as jnp
from jax.experimental import pallas as pl

_N = 2048
_K = 16
_NT = 128


def _conv_body(x_ref, w_ref, b_ref, o_ref, *, relu):
    x = x_ref[0]
    y = jnp.dot(w_ref[...], x, preferred_element_type=jnp.float32) + b_ref[...]
    if relu:
        y = jnp.maximum(y, 0.0)
    o_ref[0] = y


def _conv1x1(x, w, b, relu):
    bsz, cin, n = x.shape
    cout = w.shape[0]
    return pl.pallas_call(
        functools.partial(_conv_body, relu=relu),
        grid=(bsz,),
        in_specs=[
            pl.BlockSpec((1, cin, n), lambda i: (i, 0, 0)),
            pl.BlockSpec((cout, cin), lambda i: (0, 0)),
            pl.BlockSpec((cout, 1), lambda i: (0, 0)),
        ],
        out_specs=pl.BlockSpec((1, cout, n), lambda i: (i, 0, 0)),
        out_shape=jax.ShapeDtypeStruct((bsz, cout, n), jnp.float32),
    )(x, w, b)


def _edge_body(x_ref, wa_ref, wn_ref, w1a_ref, w1b_ref, w2a_ref, w2b_ref,
               w2c_ref, o_ref):
    t = pl.program_id(1)
    x = x_ref[0]
    xt = x_ref[0, :, pl.ds(t * _NT, _NT)]

    sq = jnp.sum(x * x, axis=0, keepdims=True)
    sqt = jnp.sum(xt * xt, axis=0)[:, None]
    prod = jnp.dot(xt.T, x, preferred_element_type=jnp.float32)
    d = sqt + sq - 2.0 * prod

    col = jax.lax.broadcasted_iota(jnp.int32, (_NT, _N), 1)
    row_g = jax.lax.broadcasted_iota(jnp.int32, (_NT, _N), 0) + t * _NT
    d = jnp.where(col == row_g, jnp.inf, d)

    g = jnp.dot(wn_ref[...], x, preferred_element_type=jnp.float32)
    gt = g.T

    gathered = []
    for _ in range(_K):
        m = jnp.min(d, axis=1, keepdims=True)
        cand = jnp.where(d == m, col, _N)
        amin = jnp.min(cand, axis=1, keepdims=True)
        onehot = (col == amin).astype(jnp.float32)
        gathered.append(
            jnp.dot(onehot, gt, preferred_element_type=jnp.float32).T)
        d = jnp.where(col == amin, jnp.inf, d)
    ge = jnp.stack(gathered, axis=1)
    ge = ge.reshape(12, _K * _NT)

    def rep(a):
        return jnp.broadcast_to(a[:, None, :], (12, _K, _NT)).reshape(
            12, _K * _NT)

    a0 = jnp.dot(wa_ref[..., :24], xt, preferred_element_type=jnp.float32) \
        + wa_ref[..., 24:25]
    c1 = jnp.dot(w1b_ref[..., :24], xt, preferred_element_type=jnp.float32) \
        + w1b_ref[..., 24:25]
    c2 = jnp.dot(w2c_ref[..., :24], xt, preferred_element_type=jnp.float32) \
        + w2c_ref[..., 24:25]

    r0 = jnp.maximum(ge + rep(a0), 0.0)
    h1 = jnp.dot(w1a_ref[...], r0, preferred_element_type=jnp.float32) + rep(c1)
    r1 = jnp.maximum(h1, 0.0)
    h2 = (jnp.dot(w2a_ref[...], r1, preferred_element_type=jnp.float32)
          + jnp.dot(w2b_ref[...], r0, preferred_element_type=jnp.float32)
          + rep(c2))

    mh2 = jnp.max(h2.reshape(12, _K, _NT), axis=1)
    mr1 = jnp.max(r1.reshape(12, _K, _NT), axis=1)
    mr0 = jnp.max(r0.reshape(12, _K, _NT), axis=1)
    o_ref[0] = jnp.concatenate([mh2, mr1, mr0, xt], axis=0)


def _edge_conv(x, mlps):
    (w0, b0), (w1, b1), (w2, b2) = mlps
    w0c, w0n = w0[:, :24], w0[:, 24:]
    wa = jnp.concatenate([w0c - w0n, b0[:, None]], axis=1)
    w1a = w1[:, :12]
    w1b = jnp.concatenate([w1[:, 12:], b1[:, None]], axis=1)
    w2a, w2b = w2[:, :12], w2[:, 12:24]
    w2c = jnp.concatenate([w2[:, 24:], b2[:, None]], axis=1)

    bsz = x.shape[0]
    wspec = lambda shape: pl.BlockSpec(shape, lambda i, j: (0, 0))
    return pl.pallas_call(
        _edge_body,
        grid=(bsz, _N // _NT),
        in_specs=[
            pl.BlockSpec((1, 24, _N), lambda i, j: (i, 0, 0)),
            wspec((12, 25)), pl.BlockSpec((12, 24), lambda i, j: (0, 0)),
            pl.BlockSpec((12, 12), lambda i, j: (0, 0)), wspec((12, 25)),
            pl.BlockSpec((12, 12), lambda i, j: (0, 0)),
            pl.BlockSpec((12, 12), lambda i, j: (0, 0)), wspec((12, 25)),
        ],
        out_specs=pl.BlockSpec((1, 60, _NT), lambda i, j: (i, 0, j)),
        out_shape=jax.ShapeDtypeStruct((bsz, 60, _N), jnp.float32),
    )(x, wa, w0n, w1a, w1b, w2a, w2b, w2c)


def _head_body(x_ref, base_ref, w1_ref, b1_ref, w2_ref, b2_ref, wf1_ref,
               bf1_ref, wf2_ref, bf2_ref, o_ref):
    x = x_ref[0]
    m = jnp.dot(w1_ref[..., :264], x,
                preferred_element_type=jnp.float32) + b1_ref[...]
    cw = w1_ref[..., 264:265]
    base = base_ref[0]
    for r, code in enumerate((-0.2, 0.2)):
        u = jnp.maximum(m + cw * code, 0.0)
        v = jnp.maximum(
            jnp.dot(w2_ref[...], u, preferred_element_type=jnp.float32)
            + b2_ref[...], 0.0)
        w = jnp.maximum(
            jnp.dot(wf1_ref[...], v, preferred_element_type=jnp.float32)
            + bf1_ref[...], 0.0)
        o = jnp.dot(wf2_ref[...], w,
                    preferred_element_type=jnp.float32) + bf2_ref[...]
        o_ref[0, :, :, r] = o + base


def _head(x, base, params):
    (w1, b1), (w2, b2) = params['up1'], params['up2']
    (wf1, bf1), (wf2, bf2) = params['fc1'], params['fc2']
    bsz, c, n = x.shape
    ws = lambda shape: pl.BlockSpec(shape, lambda i: (0, 0))
    out = pl.pallas_call(
        _head_body,
        grid=(bsz,),
        in_specs=[
            pl.BlockSpec((1, c, n), lambda i: (i, 0, 0)),
            pl.BlockSpec((1, 3, n), lambda i: (i, 0, 0)),
            ws((128, 265)), ws((128, 1)), ws((128, 128)), ws((128, 1)),
            ws((64, 128)), ws((64, 1)), ws((3, 64)), ws((3, 1)),
        ],
        out_specs=pl.BlockSpec((1, 3, n, 2), lambda i: (i, 0, 0, 0)),
        out_shape=jax.ShapeDtypeStruct((bsz, 3, n, 2), jnp.float32),
    )(x, base, w1, b1[:, None], w2, b2[:, None], wf1, bf1[:, None], wf2,
      bf2[:, None])
    return out.reshape(bsz, 3, 2 * n)


@jax.jit
def kernel(xyz, xyz_normalized, params):
    x = _conv1x1(xyz_normalized, params['layer0'][0],
                 params['layer0'][1][:, None], relu=False)
    y = _edge_conv(x, params['layer1'])
    x = jnp.concatenate([y, x], axis=1)
    for prep, lay in (('layer2_prep', 'layer2'), ('layer3_prep', 'layer3'),
                      ('layer4_prep', 'layer4')):
        w, b = params[prep]
        xp = _conv1x1(x, w, b[:, None], relu=True)
        y = _edge_conv(xp, params[lay])
        x = jnp.concatenate([y, x], axis=1)
    out = _head(x, xyz_normalized, params)
    return out, x


# speedup vs baseline: 18.8602x; 1.6282x over previous
import functools

import jax
import jax.numpy as jnp
from jax.experimental import pallas as pl

_N = 2048
_K = 16
_NT = 256


def _conv_body(x_ref, w_ref, b_ref, o_ref, *, relu):
    x = x_ref[0]
    y = jnp.dot(w_ref[...], x, preferred_element_type=jnp.float32) + b_ref[...]
    if relu:
        y = jnp.maximum(y, 0.0)
    o_ref[0] = y


def _conv1x1(x, w, b, relu):
    bsz, cin, n = x.shape
    cout = w.shape[0]
    return pl.pallas_call(
        functools.partial(_conv_body, relu=relu),
        grid=(bsz,),
        in_specs=[
            pl.BlockSpec((1, cin, n), lambda i: (i, 0, 0)),
            pl.BlockSpec((cout, cin), lambda i: (0, 0)),
            pl.BlockSpec((cout, 1), lambda i: (0, 0)),
        ],
        out_specs=pl.BlockSpec((1, cout, n), lambda i: (i, 0, 0)),
        out_shape=jax.ShapeDtypeStruct((bsz, cout, n), jnp.float32),
    )(x, w, b)


def _edge_body(x_ref, wa_ref, wn_ref, w1a_ref, w1b_ref, w2a_ref, w2b_ref,
               w2c_ref, o_ref):
    t = pl.program_id(1)
    x = x_ref[0]
    xt = x_ref[0, :, pl.ds(t * _NT, _NT)]

    sq = jnp.sum(x * x, axis=0, keepdims=True)
    sqt = jnp.sum(xt * xt, axis=0)[:, None]
    prod = jnp.dot(xt.T, x, preferred_element_type=jnp.float32)
    d = sqt + sq - 2.0 * prod

    col = jax.lax.broadcasted_iota(jnp.int32, (_NT, _N), 1)
    row_g = jax.lax.broadcasted_iota(jnp.int32, (_NT, _N), 0) + t * _NT
    d = jnp.where(col == row_g, jnp.inf, d)

    g = jnp.dot(wn_ref[...], x, preferred_element_type=jnp.float32)
    gt = g.T

    bits = jax.lax.bitcast_convert_type(d, jnp.int32)
    enc = jnp.bitwise_or(jnp.bitwise_and(bits, jnp.int32(~0x7FF)), col)

    gathered = []
    for _ in range(_K):
        amin = jnp.min(enc, axis=1, keepdims=True)
        hit = enc == amin
        onehot = hit.astype(jnp.float32)
        gathered.append(
            jnp.dot(onehot, gt, preferred_element_type=jnp.float32).T)
        enc = jnp.where(hit, jnp.int32(0x7FFFFFFF), enc)
    ge = jnp.stack(gathered, axis=1)
    ge = ge.reshape(12, _K * _NT)

    def rep(a):
        return jnp.broadcast_to(a[:, None, :], (12, _K, _NT)).reshape(
            12, _K * _NT)

    a0 = jnp.dot(wa_ref[..., :24], xt, preferred_element_type=jnp.float32) \
        + wa_ref[..., 24:25]
    c1 = jnp.dot(w1b_ref[..., :24], xt, preferred_element_type=jnp.float32) \
        + w1b_ref[..., 24:25]
    c2 = jnp.dot(w2c_ref[..., :24], xt, preferred_element_type=jnp.float32) \
        + w2c_ref[..., 24:25]

    r0 = jnp.maximum(ge + rep(a0), 0.0)
    h1 = jnp.dot(w1a_ref[...], r0, preferred_element_type=jnp.float32) + rep(c1)
    r1 = jnp.maximum(h1, 0.0)
    h2 = (jnp.dot(w2a_ref[...], r1, preferred_element_type=jnp.float32)
          + jnp.dot(w2b_ref[...], r0, preferred_element_type=jnp.float32)
          + rep(c2))

    mh2 = jnp.max(h2.reshape(12, _K, _NT), axis=1)
    mr1 = jnp.max(r1.reshape(12, _K, _NT), axis=1)
    mr0 = jnp.max(r0.reshape(12, _K, _NT), axis=1)
    o_ref[0] = jnp.concatenate([mh2, mr1, mr0, xt], axis=0)


def _edge_conv(x, mlps):
    (w0, b0), (w1, b1), (w2, b2) = mlps
    w0c, w0n = w0[:, :24], w0[:, 24:]
    wa = jnp.concatenate([w0c - w0n, b0[:, None]], axis=1)
    w1a = w1[:, :12]
    w1b = jnp.concatenate([w1[:, 12:], b1[:, None]], axis=1)
    w2a, w2b = w2[:, :12], w2[:, 12:24]
    w2c = jnp.concatenate([w2[:, 24:], b2[:, None]], axis=1)

    bsz = x.shape[0]
    wspec = lambda shape: pl.BlockSpec(shape, lambda i, j: (0, 0))
    return pl.pallas_call(
        _edge_body,
        grid=(bsz, _N // _NT),
        in_specs=[
            pl.BlockSpec((1, 24, _N), lambda i, j: (i, 0, 0)),
            wspec((12, 25)), pl.BlockSpec((12, 24), lambda i, j: (0, 0)),
            pl.BlockSpec((12, 12), lambda i, j: (0, 0)), wspec((12, 25)),
            pl.BlockSpec((12, 12), lambda i, j: (0, 0)),
            pl.BlockSpec((12, 12), lambda i, j: (0, 0)), wspec((12, 25)),
        ],
        out_specs=pl.BlockSpec((1, 60, _NT), lambda i, j: (i, 0, j)),
        out_shape=jax.ShapeDtypeStruct((bsz, 60, _N), jnp.float32),
    )(x, wa, w0n, w1a, w1b, w2a, w2b, w2c)


def _head_body(x_ref, base_ref, w1_ref, b1_ref, w2_ref, b2_ref, wf1_ref,
               bf1_ref, wf2_ref, bf2_ref, o_ref):
    x = x_ref[0]
    m = jnp.dot(w1_ref[..., :264], x,
                preferred_element_type=jnp.float32) + b1_ref[...]
    cw = w1_ref[..., 264:265]
    base = base_ref[0]
    for r, code in enumerate((-0.2, 0.2)):
        u = jnp.maximum(m + cw * code, 0.0)
        v = jnp.maximum(
            jnp.dot(w2_ref[...], u, preferred_element_type=jnp.float32)
            + b2_ref[...], 0.0)
        w = jnp.maximum(
            jnp.dot(wf1_ref[...], v, preferred_element_type=jnp.float32)
            + bf1_ref[...], 0.0)
        o = jnp.dot(wf2_ref[...], w,
                    preferred_element_type=jnp.float32) + bf2_ref[...]
        o_ref[0, :, :, r] = o + base


def _head(x, base, params):
    (w1, b1), (w2, b2) = params['up1'], params['up2']
    (wf1, bf1), (wf2, bf2) = params['fc1'], params['fc2']
    bsz, c, n = x.shape
    ws = lambda shape: pl.BlockSpec(shape, lambda i: (0, 0))
    out = pl.pallas_call(
        _head_body,
        grid=(bsz,),
        in_specs=[
            pl.BlockSpec((1, c, n), lambda i: (i, 0, 0)),
            pl.BlockSpec((1, 3, n), lambda i: (i, 0, 0)),
            ws((128, 265)), ws((128, 1)), ws((128, 128)), ws((128, 1)),
            ws((64, 128)), ws((64, 1)), ws((3, 64)), ws((3, 1)),
        ],
        out_specs=pl.BlockSpec((1, 3, n, 2), lambda i: (i, 0, 0, 0)),
        out_shape=jax.ShapeDtypeStruct((bsz, 3, n, 2), jnp.float32),
    )(x, base, w1, b1[:, None], w2, b2[:, None], wf1, bf1[:, None], wf2,
      bf2[:, None])
    return out.reshape(bsz, 3, 2 * n)


@jax.jit
def kernel(xyz, xyz_normalized, params):
    x = _conv1x1(xyz_normalized, params['layer0'][0],
                 params['layer0'][1][:, None], relu=False)
    y = _edge_conv(x, params['layer1'])
    x = jnp.concatenate([y, x], axis=1)
    for prep, lay in (('layer2_prep', 'layer2'), ('layer3_prep', 'layer3'),
                      ('layer4_prep', 'layer4')):
        w, b = params[prep]
        xp = _conv1x1(x, w, b[:, None], relu=True)
        y = _edge_conv(xp, params[lay])
        x = jnp.concatenate([y, x], axis=1)
    out = _head(x, xyz_normalized, params)
    return out, x


# speedup vs baseline: 21.3035x; 1.1295x over previous
import functools

import jax
import jax.numpy as jnp
from jax.experimental import pallas as pl

_N = 2048
_K = 16
_NT = 256


def _conv_body(x_ref, w_ref, b_ref, o_ref, *, relu):
    x = x_ref[0]
    y = jnp.dot(w_ref[...], x, preferred_element_type=jnp.float32) + b_ref[...]
    if relu:
        y = jnp.maximum(y, 0.0)
    o_ref[0] = y


def _conv1x1(x, w, b, relu):
    bsz, cin, n = x.shape
    cout = w.shape[0]
    return pl.pallas_call(
        functools.partial(_conv_body, relu=relu),
        grid=(bsz,),
        in_specs=[
            pl.BlockSpec((1, cin, n), lambda i: (i, 0, 0)),
            pl.BlockSpec((cout, cin), lambda i: (0, 0)),
            pl.BlockSpec((cout, 1), lambda i: (0, 0)),
        ],
        out_specs=pl.BlockSpec((1, cout, n), lambda i: (i, 0, 0)),
        out_shape=jax.ShapeDtypeStruct((bsz, cout, n), jnp.float32),
    )(x, w, b)


def _edge_body(x_ref, wa_ref, wn_ref, w1a_ref, w1b_ref, w2a_ref, w2b_ref,
               w2c_ref, o_ref):
    t = pl.program_id(1)
    x = x_ref[0]
    xt = x_ref[0, :, pl.ds(t * _NT, _NT)]

    sq = jnp.sum(x * x, axis=0, keepdims=True)
    sqt = jnp.sum(xt * xt, axis=0)[:, None]
    prod = jnp.dot(xt.T, x, preferred_element_type=jnp.float32)
    d = sqt + sq - 2.0 * prod

    col = jax.lax.broadcasted_iota(jnp.int32, (_NT, _N), 1)
    row_g = jax.lax.broadcasted_iota(jnp.int32, (_NT, _N), 0) + t * _NT
    d = jnp.where(col == row_g, jnp.float32(3e38), d)

    g = jnp.dot(wn_ref[...], x, preferred_element_type=jnp.float32)
    gt = g.T

    bits = jax.lax.bitcast_convert_type(d, jnp.int32)
    enc = jax.lax.bitcast_convert_type(
        jnp.bitwise_or(jnp.bitwise_and(bits, jnp.int32(~0x7FF)), col),
        jnp.float32)

    gathered = []
    for _ in range(_K):
        amin = jnp.min(enc, axis=1, keepdims=True)
        hit = enc == amin
        onehot = hit.astype(jnp.float32)
        gathered.append(
            jnp.dot(onehot, gt, preferred_element_type=jnp.float32).T)
        enc = jnp.where(hit, jnp.float32(jnp.inf), enc)
    ge = jnp.stack(gathered, axis=1)
    ge = ge.reshape(12, _K * _NT)

    def rep(a):
        return jnp.broadcast_to(a[:, None, :], (12, _K, _NT)).reshape(
            12, _K * _NT)

    a0 = jnp.dot(wa_ref[..., :24], xt, preferred_element_type=jnp.float32) \
        + wa_ref[..., 24:25]
    c1 = jnp.dot(w1b_ref[..., :24], xt, preferred_element_type=jnp.float32) \
        + w1b_ref[..., 24:25]
    c2 = jnp.dot(w2c_ref[..., :24], xt, preferred_element_type=jnp.float32) \
        + w2c_ref[..., 24:25]

    r0 = jnp.maximum(ge + rep(a0), 0.0)
    h1 = jnp.dot(w1a_ref[...], r0, preferred_element_type=jnp.float32) + rep(c1)
    r1 = jnp.maximum(h1, 0.0)
    h2 = (jnp.dot(w2a_ref[...], r1, preferred_element_type=jnp.float32)
          + jnp.dot(w2b_ref[...], r0, preferred_element_type=jnp.float32)
          + rep(c2))

    mh2 = jnp.max(h2.reshape(12, _K, _NT), axis=1)
    mr1 = jnp.max(r1.reshape(12, _K, _NT), axis=1)
    mr0 = jnp.max(r0.reshape(12, _K, _NT), axis=1)
    o_ref[0] = jnp.concatenate([mh2, mr1, mr0, xt], axis=0)


def _edge_conv(x, mlps):
    (w0, b0), (w1, b1), (w2, b2) = mlps
    w0c, w0n = w0[:, :24], w0[:, 24:]
    wa = jnp.concatenate([w0c - w0n, b0[:, None]], axis=1)
    w1a = w1[:, :12]
    w1b = jnp.concatenate([w1[:, 12:], b1[:, None]], axis=1)
    w2a, w2b = w2[:, :12], w2[:, 12:24]
    w2c = jnp.concatenate([w2[:, 24:], b2[:, None]], axis=1)

    bsz = x.shape[0]
    wspec = lambda shape: pl.BlockSpec(shape, lambda i, j: (0, 0))
    return pl.pallas_call(
        _edge_body,
        grid=(bsz, _N // _NT),
        in_specs=[
            pl.BlockSpec((1, 24, _N), lambda i, j: (i, 0, 0)),
            wspec((12, 25)), pl.BlockSpec((12, 24), lambda i, j: (0, 0)),
            pl.BlockSpec((12, 12), lambda i, j: (0, 0)), wspec((12, 25)),
            pl.BlockSpec((12, 12), lambda i, j: (0, 0)),
            pl.BlockSpec((12, 12), lambda i, j: (0, 0)), wspec((12, 25)),
        ],
        out_specs=pl.BlockSpec((1, 60, _NT), lambda i, j: (i, 0, j)),
        out_shape=jax.ShapeDtypeStruct((bsz, 60, _N), jnp.float32),
    )(x, wa, w0n, w1a, w1b, w2a, w2b, w2c)


def _head_body(x_ref, base_ref, w1_ref, b1_ref, w2_ref, b2_ref, wf1_ref,
               bf1_ref, wf2_ref, bf2_ref, o_ref):
    x = x_ref[0]
    m = jnp.dot(w1_ref[..., :264], x,
                preferred_element_type=jnp.float32) + b1_ref[...]
    cw = w1_ref[..., 264:265]
    base = base_ref[0]
    for r, code in enumerate((-0.2, 0.2)):
        u = jnp.maximum(m + cw * code, 0.0)
        v = jnp.maximum(
            jnp.dot(w2_ref[...], u, preferred_element_type=jnp.float32)
            + b2_ref[...], 0.0)
        w = jnp.maximum(
            jnp.dot(wf1_ref[...], v, preferred_element_type=jnp.float32)
            + bf1_ref[...], 0.0)
        o = jnp.dot(wf2_ref[...], w,
                    preferred_element_type=jnp.float32) + bf2_ref[...]
        o_ref[0, :, :, r] = o + base


def _head(x, base, params):
    (w1, b1), (w2, b2) = params['up1'], params['up2']
    (wf1, bf1), (wf2, bf2) = params['fc1'], params['fc2']
    bsz, c, n = x.shape
    ws = lambda shape: pl.BlockSpec(shape, lambda i: (0, 0))
    out = pl.pallas_call(
        _head_body,
        grid=(bsz,),
        in_specs=[
            pl.BlockSpec((1, c, n), lambda i: (i, 0, 0)),
            pl.BlockSpec((1, 3, n), lambda i: (i, 0, 0)),
            ws((128, 265)), ws((128, 1)), ws((128, 128)), ws((128, 1)),
            ws((64, 128)), ws((64, 1)), ws((3, 64)), ws((3, 1)),
        ],
        out_specs=pl.BlockSpec((1, 3, n, 2), lambda i: (i, 0, 0, 0)),
        out_shape=jax.ShapeDtypeStruct((bsz, 3, n, 2), jnp.float32),
    )(x, base, w1, b1[:, None], w2, b2[:, None], wf1, bf1[:, None], wf2,
      bf2[:, None])
    return out.reshape(bsz, 3, 2 * n)


@jax.jit
def kernel(xyz, xyz_normalized, params):
    x = _conv1x1(xyz_normalized, params['layer0'][0],
                 params['layer0'][1][:, None], relu=False)
    y = _edge_conv(x, params['layer1'])
    x = jnp.concatenate([y, x], axis=1)
    for prep, lay in (('layer2_prep', 'layer2'), ('layer3_prep', 'layer3'),
                      ('layer4_prep', 'layer4')):
        w, b = params[prep]
        xp = _conv1x1(x, w, b[:, None], relu=True)
        y = _edge_conv(xp, params[lay])
        x = jnp.concatenate([y, x], axis=1)
    out = _head(x, xyz_normalized, params)
    return out, x


# speedup vs baseline: 23.7050x; 1.1127x over previous
import functools

import jax
import jax.numpy as jnp
from jax.experimental import pallas as pl
from jax.experimental.pallas import tpu as pltpu
from jax.experimental.pallas import tpu_sc as plsc

_N = 2048
_K = 16
_B = 8
_NT = 256
_NT2 = 256


def _conv_body(x_ref, w_ref, b_ref, wn_ref, o_ref, g_ref, *, relu):
    x = x_ref[0]
    y = jnp.dot(w_ref[...], x, preferred_element_type=jnp.float32) + b_ref[...]
    if relu:
        y = jnp.maximum(y, 0.0)
    o_ref[0] = y
    g = jnp.dot(wn_ref[...], y, preferred_element_type=jnp.float32)
    g_ref[0] = jnp.concatenate(
        [g, jnp.zeros((4, g.shape[1]), jnp.float32)], axis=0).T


def _conv1x1(x, w, b, wn, relu):
    bsz, cin, n = x.shape
    cout = w.shape[0]
    return pl.pallas_call(
        functools.partial(_conv_body, relu=relu),
        grid=(bsz,),
        in_specs=[
            pl.BlockSpec((1, cin, n), lambda i: (i, 0, 0)),
            pl.BlockSpec((cout, cin), lambda i: (0, 0)),
            pl.BlockSpec((cout, 1), lambda i: (0, 0)),
            pl.BlockSpec((12, cout), lambda i: (0, 0)),
        ],
        out_specs=[
            pl.BlockSpec((1, cout, n), lambda i: (i, 0, 0)),
            pl.BlockSpec((1, n, 16), lambda i: (i, 0, 0)),
        ],
        out_shape=[
            jax.ShapeDtypeStruct((bsz, cout, n), jnp.float32),
            jax.ShapeDtypeStruct((bsz, n, 16), jnp.float32),
        ],
    )(x, w, b, wn)


def _topk_body(x_ref, o_ref):
    t = pl.program_id(1)
    x = x_ref[0]
    xt = x_ref[0, :, pl.ds(t * _NT, _NT)]

    sq = jnp.sum(x * x, axis=0, keepdims=True)
    sqt = jnp.sum(xt * xt, axis=0)[:, None]
    prod = jnp.dot(xt.T, x, preferred_element_type=jnp.float32)
    d = sqt + sq - 2.0 * prod

    col = jax.lax.broadcasted_iota(jnp.int32, (_NT, _N), 1)
    row_g = jax.lax.broadcasted_iota(jnp.int32, (_NT, _N), 0) + t * _NT
    d = jnp.where(col == row_g, jnp.float32(3e38), d)

    bits = jax.lax.bitcast_convert_type(d, jnp.int32)
    enc = jax.lax.bitcast_convert_type(
        jnp.bitwise_or(jnp.bitwise_and(bits, jnp.int32(~0x7FF)), col),
        jnp.float32)

    prev = jnp.full((_NT, 1), -jnp.inf, jnp.float32)
    cols = []
    for _ in range(_K):
        cand = jnp.where(enc > prev, enc, jnp.float32(jnp.inf))
        amin = jnp.min(cand, axis=1, keepdims=True)
        cols.append(jnp.bitwise_and(
            jax.lax.bitcast_convert_type(amin, jnp.int32), jnp.int32(2047)))
        prev = amin
    rows = jnp.concatenate(cols, axis=1)
    o_ref[0] = rows.T


def _edge_topk(x):
    return pl.pallas_call(
        _topk_body,
        grid=(_B, _N // _NT),
        in_specs=[pl.BlockSpec((1, 24, _N), lambda i, j: (i, 0, 0))],
        out_specs=pl.BlockSpec((1, _K, _NT), lambda i, j: (i, 0, j)),
        out_shape=jax.ShapeDtypeStruct((_B, _K, _N), jnp.int32),
    )(x)


def _sc_gather_body(table_hbm, idx_hbm, out_hbm, table_v, idx_v, out_v):
    wid = jax.lax.axis_index("s") * 2 + jax.lax.axis_index("c")
    b = wid // 4
    pltpu.sync_copy(table_hbm.at[pl.ds(b * _N, _N)], table_v)
    pltpu.sync_copy(idx_hbm.at[pl.ds(wid * 512, 512)], idx_v)
    lane = jax.lax.broadcasted_iota(jnp.int32, (16,), 0)
    for c in range(2):
        def body(i, carry):
            idx16 = plsc.load_gather(
                idx_v, [jnp.full((16,), c * 256, jnp.int32) + i, lane])
            iv = jnp.full((16,), i, jnp.int32)
            for f in range(12):
                fv = jnp.full((16,), f, jnp.int32)
                vals = plsc.load_gather(table_v, [idx16, fv])
                plsc.store_scatter(out_v, [fv, iv, lane], vals)
            return carry
        jax.lax.fori_loop(0, 256, body, 0)
        pltpu.sync_copy(out_v, out_hbm.at[:, wid * 2 + c])


def _sc_gather(table, idx2d):
    nrow = _B * _N * _K // 16
    return pl.kernel(
        _sc_gather_body,
        out_type=pltpu.HBM((12, 64, 256, 16), jnp.float32),
        mesh=plsc.VectorSubcoreMesh(core_axis_name="c", subcore_axis_name="s"),
        compiler_params=pltpu.CompilerParams(
            needs_layout_passes=False, use_tc_tiling_on_sc=False),
        scratch_types=[
            pltpu.VMEM((_N, 16), jnp.float32),
            pltpu.VMEM((512, 16), jnp.int32),
            pltpu.VMEM((12, 256, 16), jnp.float32),
        ],
    )(table, idx2d.reshape(nrow, 16))


def _mlp_body(x_ref, ge_ref, wa_ref, w1a_ref, w1b_ref, w2a_ref, w2b_ref,
              w2c_ref, o_ref):
    t = pl.program_id(1)
    xt = x_ref[0, :, pl.ds(t * _NT2, _NT2)]
    ge = ge_ref[:, 0].reshape(12, _K * _NT2)

    def rep(a):
        return jnp.broadcast_to(a[:, None, :], (12, _K, _NT2)).reshape(
            12, _K * _NT2)

    a0 = jnp.dot(wa_ref[..., :24], xt, preferred_element_type=jnp.float32) \
        + wa_ref[..., 24:25]
    c1 = jnp.dot(w1b_ref[..., :24], xt, preferred_element_type=jnp.float32) \
        + w1b_ref[..., 24:25]
    c2 = jnp.dot(w2c_ref[..., :24], xt, preferred_element_type=jnp.float32) \
        + w2c_ref[..., 24:25]

    r0 = jnp.maximum(ge + rep(a0), 0.0)
    h1 = jnp.dot(w1a_ref[...], r0, preferred_element_type=jnp.float32) + rep(c1)
    r1 = jnp.maximum(h1, 0.0)
    h2 = (jnp.dot(w2a_ref[...], r1, preferred_element_type=jnp.float32)
          + jnp.dot(w2b_ref[...], r0, preferred_element_type=jnp.float32)
          + rep(c2))

    mh2 = jnp.max(h2.reshape(12, _K, _NT2), axis=1)
    mr1 = jnp.max(r1.reshape(12, _K, _NT2), axis=1)
    mr0 = jnp.max(r0.reshape(12, _K, _NT2), axis=1)
    o_ref[0] = jnp.concatenate([mh2, mr1, mr0, xt], axis=0)


def _edge_mlp(x, ge, mlps):
    (w0, b0), (w1, b1), (w2, b2) = mlps
    w0c, w0n = w0[:, :24], w0[:, 24:]
    wa = jnp.concatenate([w0c - w0n, b0[:, None]], axis=1)
    w1a = w1[:, :12]
    w1b = jnp.concatenate([w1[:, 12:], b1[:, None]], axis=1)
    w2a, w2b = w2[:, :12], w2[:, 12:24]
    w2c = jnp.concatenate([w2[:, 24:], b2[:, None]], axis=1)

    wspec = lambda shape: pl.BlockSpec(shape, lambda i, j: (0, 0))
    return pl.pallas_call(
        _mlp_body,
        grid=(_B, _N // _NT2),
        in_specs=[
            pl.BlockSpec((1, 24, _N), lambda i, j: (i, 0, 0)),
            pl.BlockSpec((12, 1, _K, _NT2), lambda i, j: (0, i, 0, j)),
            wspec((12, 25)), wspec((12, 12)), wspec((12, 25)),
            wspec((12, 12)), wspec((12, 12)), wspec((12, 25)),
        ],
        out_specs=pl.BlockSpec((1, 60, _NT2), lambda i, j: (i, 0, j)),
        out_shape=jax.ShapeDtypeStruct((_B, 60, _N), jnp.float32),
    )(x, ge, wa, w1a, w1b, w2a, w2b, w2c)


def _edge_conv(x, gtab, mlps):
    idx = _edge_topk(x)
    ge = _sc_gather(gtab.reshape(_B * _N, 16),
                    idx.reshape(_B * _K * _N // 128, 128))
    return _edge_mlp(x, ge.reshape(12, _B * _K * _N).reshape(12, _B, _K, _N),
                     mlps)


def _head_body(x_ref, base_ref, w1_ref, b1_ref, w2_ref, b2_ref, wf1_ref,
               bf1_ref, wf2_ref, bf2_ref, o_ref):
    x = x_ref[0]
    m = jnp.dot(w1_ref[..., :264], x,
                preferred_element_type=jnp.float32) + b1_ref[...]
    cw = w1_ref[..., 264:265]
    base = base_ref[0]
    for r, code in enumerate((-0.2, 0.2)):
        u = jnp.maximum(m + cw * code, 0.0)
        v = jnp.maximum(
            jnp.dot(w2_ref[...], u, preferred_element_type=jnp.float32)
            + b2_ref[...], 0.0)
        w = jnp.maximum(
            jnp.dot(wf1_ref[...], v, preferred_element_type=jnp.float32)
            + bf1_ref[...], 0.0)
        o = jnp.dot(wf2_ref[...], w,
                    preferred_element_type=jnp.float32) + bf2_ref[...]
        o_ref[0, :, :, r] = o + base


def _head(x, base, params):
    (w1, b1), (w2, b2) = params['up1'], params['up2']
    (wf1, bf1), (wf2, bf2) = params['fc1'], params['fc2']
    bsz, c, n = x.shape
    ws = lambda shape: pl.BlockSpec(shape, lambda i: (0, 0))
    out = pl.pallas_call(
        _head_body,
        grid=(bsz,),
        in_specs=[
            pl.BlockSpec((1, c, n), lambda i: (i, 0, 0)),
            pl.BlockSpec((1, 3, n), lambda i: (i, 0, 0)),
            ws((128, 265)), ws((128, 1)), ws((128, 128)), ws((128, 1)),
            ws((64, 128)), ws((64, 1)), ws((3, 64)), ws((3, 1)),
        ],
        out_specs=pl.BlockSpec((1, 3, n, 2), lambda i: (i, 0, 0, 0)),
        out_shape=jax.ShapeDtypeStruct((bsz, 3, n, 2), jnp.float32),
    )(x, base, w1, b1[:, None], w2, b2[:, None], wf1, bf1[:, None], wf2,
      bf2[:, None])
    return out.reshape(bsz, 3, 2 * n)


@jax.jit
def kernel(xyz, xyz_normalized, params):
    w, b = params['layer0']
    x, gtab = _conv1x1(xyz_normalized, w, b[:, None],
                       params['layer1'][0][0][:, 24:], relu=False)
    y = _edge_conv(x, gtab, params['layer1'])
    x = jnp.concatenate([y, x], axis=1)
    for prep, lay in (('layer2_prep', 'layer2'), ('layer3_prep', 'layer3'),
                      ('layer4_prep', 'layer4')):
        w, b = params[prep]
        xp, gtab = _conv1x1(x, w, b[:, None], params[lay][0][0][:, 24:],
                            relu=True)
        y = _edge_conv(xp, gtab, params[lay])
        x = jnp.concatenate([y, x], axis=1)
    out = _head(x, xyz_normalized, params)
    return out, x
